# baseline (speedup 1.0000x reference)
import functools
import math

import jax
import jax.numpy as jnp
from jax import lax
from jax.experimental import pallas as pl
from jax.experimental.pallas import tpu as pltpu
from jax.experimental.pallas import tpu_sc as plsc

_D = 64
_B = 4096
_L = 200

_NC = 2
_NS = 16
_NW = _NC * _NS

_N = _B * _L
_IB = 128
_ROWS_PER_W = _N // _NW
_K = 1024
_J = _K // _IB
_CHUNKS = _ROWS_PER_W // _K
_SCALE = math.sqrt(_D)


def _emb_body(idx_hbm, lut_hbm, out_hbm, idx_v, rows_v, sem):
    wid = lax.axis_index("s") * _NC + lax.axis_index("c")
    row0 = wid * (_ROWS_PER_W // _IB)
    out0 = wid * _ROWS_PER_W

    def chunk(g, carry):
        pltpu.sync_copy(idx_hbm.at[pl.ds(row0 + g * _J, _J), :], idx_v)
        cps = [
            pltpu.async_copy(
                lut_hbm.at[idx_v.at[j]],
                rows_v.at[pl.ds(j * _IB, _IB), :],
                sem,
            )
            for j in range(_J)
        ]
        for cp in cps:
            cp.wait()

        def srow(i, c):
            for t in range(_D // 16):
                sl = (i, pl.ds(t * 16, 16))
                rows_v[sl] = rows_v[sl] * _SCALE
            return c

        lax.fori_loop(0, _K, srow, 0)
        pltpu.sync_copy(rows_v, out_hbm.at[pl.ds(out0 + g * _K, _K), :])
        return carry

    lax.fori_loop(0, _CHUNKS, chunk, 0)


def kernel(x, lut):
    idx2d = x.reshape(_N // _IB, _IB)
    run = functools.partial(
        pl.kernel,
        mesh=plsc.VectorSubcoreMesh(core_axis_name="c", subcore_axis_name="s"),
        out_type=jax.ShapeDtypeStruct((_N, _D), jnp.float32),
        scratch_types=[
            pltpu.VMEM((_J, _IB), jnp.int32),
            pltpu.VMEM((_K, _D), jnp.float32),
            pltpu.SemaphoreType.DMA,
        ],
        compiler_params=pltpu.CompilerParams(use_tc_tiling_on_sc=False),
    )(_emb_body)
    out = run(idx2d, lut)
    return out.reshape(_B, _L, _D)


# speedup vs baseline: 1.1067x; 1.1067x over previous
import functools
import math

import jax
import jax.numpy as jnp
from jax import lax
from jax.experimental import pallas as pl
from jax.experimental.pallas import tpu as pltpu
from jax.experimental.pallas import tpu_sc as plsc

_D = 64
_B = 4096
_L = 200

_NC = 2
_NS = 16
_NW = _NC * _NS

_N = _B * _L
_IB = 128
_ROWS_PER_W = _N // _NW
_IROWS = _ROWS_PER_W // _IB
_K = 512
_J = _K // _IB
_CHUNKS = _ROWS_PER_W // _K
_UNROLL = 8
_SCALE = math.sqrt(_D)


def _emb_body(idx_hbm, lut_hbm, out_hbm, idx_v, rows0, rows1,
              gsem0, gsem1, wsem0, wsem1):
    wid = lax.axis_index("s") * _NC + lax.axis_index("c")
    irow0 = wid * _IROWS
    out0 = wid * _ROWS_PER_W

    bufs = (rows0, rows1)
    gsems = (gsem0, gsem1)
    wsems = (wsem0, wsem1)

    pltpu.sync_copy(idx_hbm.at[pl.ds(irow0, _IROWS), :], idx_v)

    def fire_g(g, b):
        for j in range(_J):
            pltpu.make_async_copy(
                lut_hbm.at[idx_v.at[g * _J + j]],
                bufs[b].at[pl.ds(j * _IB, _IB), :],
                gsems[b],
            ).start()

    def wait_g(b):
        pltpu.make_async_copy(
            out_hbm.at[pl.ds(0, _K), :], bufs[b], gsems[b]).wait()

    def fire_w(g, b):
        pltpu.make_async_copy(
            bufs[b], out_hbm.at[pl.ds(out0 + g * _K, _K), :], wsems[b]).start()

    def wait_w(b):
        pltpu.make_async_copy(
            bufs[b], out_hbm.at[pl.ds(out0, _K), :], wsems[b]).wait()

    def scale(b):
        r = bufs[b]

        def body(i, c):
            for u in range(_UNROLL):
                for t in range(_D // 16):
                    sl = (i * _UNROLL + u, pl.ds(t * 16, 16))
                    r[sl] = r[sl] * _SCALE
            return c

        lax.fori_loop(0, _K // _UNROLL, body, 0)

    fire_g(0, 0)
    fire_g(1, 1)
    wait_g(0)
    scale(0)
    fire_w(0, 0)

    def pair(p, c):
        g = 1 + 2 * p
        wait_w(0)
        fire_g(g + 1, 0)
        wait_g(1)
        scale(1)
        fire_w(g, 1)
        wait_w(1)
        fire_g(g + 2, 1)
        wait_g(0)
        scale(0)
        fire_w(g + 1, 0)
        return c

    lax.fori_loop(0, (_CHUNKS - 2) // 2, pair, 0)

    wait_g(1)
    scale(1)
    fire_w(_CHUNKS - 1, 1)
    wait_w(0)
    wait_w(1)


def kernel(x, lut):
    idx2d = x.reshape(_N // _IB, _IB)
    run = functools.partial(
        pl.kernel,
        mesh=plsc.VectorSubcoreMesh(core_axis_name="c", subcore_axis_name="s"),
        out_type=jax.ShapeDtypeStruct((_N, _D), jnp.float32),
        scratch_types=[
            pltpu.VMEM((_IROWS, _IB), jnp.int32),
            pltpu.VMEM((_K, _D), jnp.float32),
            pltpu.VMEM((_K, _D), jnp.float32),
            pltpu.SemaphoreType.DMA,
            pltpu.SemaphoreType.DMA,
            pltpu.SemaphoreType.DMA,
            pltpu.SemaphoreType.DMA,
        ],
        compiler_params=pltpu.CompilerParams(use_tc_tiling_on_sc=False),
    )(_emb_body)
    out = run(idx2d, lut)
    return out.reshape(_B, _L, _D)


# speedup vs baseline: 1.1087x; 1.0018x over previous
import functools
import math

import jax
import jax.numpy as jnp
from jax import lax
from jax.experimental import pallas as pl
from jax.experimental.pallas import tpu as pltpu
from jax.experimental.pallas import tpu_sc as plsc

_D = 64
_B = 4096
_L = 200

_NC = 2
_NS = 16
_NW = _NC * _NS

_N = _B * _L
_IB = 128
_ROWS_PER_W = _N // _NW
_IROWS = _ROWS_PER_W // _IB
_K = 512
_J = _K // _IB
_CHUNKS = _ROWS_PER_W // _K
_UNROLL = 8
_SCALE = math.sqrt(_D)


def _emb_body(idx_hbm, lut_hbm, out_hbm, idx_v, rows0, rows1,
              gsem0, gsem1, wsem0, wsem1):
    wid = lax.axis_index("s") * _NC + lax.axis_index("c")
    irow0 = wid * _IROWS
    out0 = wid * _ROWS_PER_W

    bufs = (rows0, rows1)
    gsems = (gsem0, gsem1)
    wsems = (wsem0, wsem1)

    pltpu.sync_copy(idx_hbm.at[pl.ds(irow0, _IROWS), :], idx_v)

    def fire_g(g, b):
        for j in range(_J):
            pltpu.make_async_copy(
                lut_hbm.at[idx_v.at[g * _J + j]],
                bufs[b].at[pl.ds(j * _IB, _IB), :],
                gsems[b],
            ).start()

    def wait_g(b):
        pltpu.make_async_copy(
            out_hbm.at[pl.ds(0, _K), :], bufs[b], gsems[b]).wait()

    def fire_w(g, b):
        pltpu.make_async_copy(
            bufs[b], out_hbm.at[pl.ds(out0 + g * _K, _K), :], wsems[b]).start()

    def wait_w(b):
        pltpu.make_async_copy(
            bufs[b], out_hbm.at[pl.ds(out0, _K), :], wsems[b]).wait()

    def scale(b):
        r = bufs[b]

        @plsc.parallel_loop(0, _K, step=1, unroll=_UNROLL)
        def body(i):
            for t in range(_D // 16):
                sl = (i, pl.ds(t * 16, 16))
                r[sl] = r[sl] * _SCALE

    fire_g(0, 0)
    fire_g(1, 1)
    wait_g(0)
    scale(0)
    fire_w(0, 0)

    def pair(p, c):
        g = 1 + 2 * p
        wait_w(0)
        fire_g(g + 1, 0)
        wait_g(1)
        scale(1)
        fire_w(g, 1)
        wait_w(1)
        fire_g(g + 2, 1)
        wait_g(0)
        scale(0)
        fire_w(g + 1, 0)
        return c

    lax.fori_loop(0, (_CHUNKS - 2) // 2, pair, 0)

    wait_g(1)
    scale(1)
    fire_w(_CHUNKS - 1, 1)
    wait_w(0)
    wait_w(1)


def kernel(x, lut):
    idx2d = x.reshape(_N // _IB, _IB)
    run = functools.partial(
        pl.kernel,
        mesh=plsc.VectorSubcoreMesh(core_axis_name="c", subcore_axis_name="s"),
        out_type=jax.ShapeDtypeStruct((_N, _D), jnp.float32),
        scratch_types=[
            pltpu.VMEM((_IROWS, _IB), jnp.int32),
            pltpu.VMEM((_K, _D), jnp.float32),
            pltpu.VMEM((_K, _D), jnp.float32),
            pltpu.SemaphoreType.DMA,
            pltpu.SemaphoreType.DMA,
            pltpu.SemaphoreType.DMA,
            pltpu.SemaphoreType.DMA,
        ],
        compiler_params=pltpu.CompilerParams(use_tc_tiling_on_sc=False),
    )(_emb_body)
    out = run(idx2d, lut)
    return out.reshape(_B, _L, _D)


# speedup vs baseline: 1.1966x; 1.0793x over previous
import functools
import math

import jax
import jax.numpy as jnp
from jax import lax
from jax.experimental import pallas as pl
from jax.experimental.pallas import tpu as pltpu
from jax.experimental.pallas import tpu_sc as plsc

_D = 64
_B = 4096
_L = 200

_NC = 2
_NS = 16
_NW = _NC * _NS

_N = _B * _L
_IB = 128
_ROWS_PER_W = _N // _NW
_K = 128
_J = _K // _IB
_CHUNKS = _ROWS_PER_W // _K
_UNROLL = 8
_SCALE = math.sqrt(_D)


def _emb_body(idx_hbm, lut_hbm, out_hbm, idx0, idx1, rows0, rows1,
              stg0, stg1, isem0, isem1, gsem0, gsem1, wsem0, wsem1):
    wid = lax.axis_index("s") * _NC + lax.axis_index("c")
    base = wid * _ROWS_PER_W

    idxs = (idx0, idx1)
    rows = (rows0, rows1)
    stgs = (stg0, stg1)
    isems = (isem0, isem1)
    gsems = (gsem0, gsem1)
    wsems = (wsem0, wsem1)

    def fire_i(g, b):
        pltpu.make_async_copy(
            idx_hbm.at[pl.ds(base + g * _K, _K)], idxs[b], isems[b]).start()

    def wait_i(b):
        pltpu.make_async_copy(
            idx_hbm.at[pl.ds(0, _K)], idxs[b], isems[b]).wait()

    def fire_g(b):
        for j in range(_J):
            pltpu.make_async_copy(
                lut_hbm.at[idxs[b].at[pl.ds(j * _IB, _IB)]],
                rows[b].at[pl.ds(j * _IB, _IB), :],
                gsems[b],
            ).start()

    def wait_g(b):
        pltpu.make_async_copy(
            lut_hbm.at[pl.ds(0, _K), :], rows[b], gsems[b]).wait()

    def fire_w(g, b):
        pltpu.make_async_copy(
            stgs[b], out_hbm.at[pl.ds(base + g * _K, _K), :], wsems[b]).start()

    def wait_w(b):
        pltpu.make_async_copy(
            stgs[b], out_hbm.at[pl.ds(0, _K), :], wsems[b]).wait()

    def scale(b):
        r = rows[b]
        s = stgs[b]

        @plsc.parallel_loop(0, _K, step=1, unroll=_UNROLL)
        def body(i):
            for t in range(_D // 16):
                s[i, pl.ds(16 * t, 16)] = r[i, pl.ds(16 * t, 16)] * _SCALE

    fire_i(0, 0)
    fire_i(1, 1)
    wait_i(0)
    fire_g(0)
    wait_i(1)
    fire_g(1)
    wait_g(0)
    scale(0)
    fire_w(0, 0)

    def pair(p, c):
        g = 1 + 2 * p
        fire_i(g + 1, 0)
        wait_g(1)
        scale(1)
        fire_w(g, 1)
        fire_i(g + 2, 1)
        wait_i(0)
        wait_w(0)
        fire_g(0)
        wait_g(0)
        scale(0)
        fire_w(g + 1, 0)
        wait_i(1)
        wait_w(1)
        fire_g(1)
        return c

    lax.fori_loop(0, (_CHUNKS - 2) // 2, pair, 0)

    wait_g(1)
    scale(1)
    fire_w(_CHUNKS - 1, 1)
    wait_w(0)
    wait_w(1)


def kernel(x, lut):
    idx1d = x.reshape(_N)
    lut_pad = jnp.pad(lut, ((0, 0), (0, 128 - _D)))
    run = functools.partial(
        pl.kernel,
        mesh=plsc.VectorSubcoreMesh(core_axis_name="c", subcore_axis_name="s"),
        out_type=jax.ShapeDtypeStruct((_N, _D), jnp.float32),
        scratch_types=[
            pltpu.VMEM((_K,), jnp.int32),
            pltpu.VMEM((_K,), jnp.int32),
            pltpu.VMEM((_K, 128), jnp.float32),
            pltpu.VMEM((_K, 128), jnp.float32),
            pltpu.VMEM((_K, _D), jnp.float32),
            pltpu.VMEM((_K, _D), jnp.float32),
            pltpu.SemaphoreType.DMA,
            pltpu.SemaphoreType.DMA,
            pltpu.SemaphoreType.DMA,
            pltpu.SemaphoreType.DMA,
            pltpu.SemaphoreType.DMA,
            pltpu.SemaphoreType.DMA,
        ],
        compiler_params=pltpu.CompilerParams(use_tc_tiling_on_sc=True),
    )(_emb_body)
    out = run(idx1d, lut_pad)
    return out.reshape(_B, _L, _D)


# speedup vs baseline: 1.3506x; 1.1287x over previous
import functools
import math

import jax
import jax.numpy as jnp
from jax import lax
from jax.experimental import pallas as pl
from jax.experimental.pallas import tpu as pltpu
from jax.experimental.pallas import tpu_sc as plsc

_D = 64
_B = 4096
_L = 200

_NC = 2
_NS = 16
_NW = _NC * _NS

_N = _B * _L
_IB = 128
_ROWS_PER_W = _N // _NW
_K = 128
_J = _K // _IB
_CHUNKS = _ROWS_PER_W // _K
_UNROLL = 8
_SCALE = math.sqrt(_D)


def _emb_body(idx_hbm, lut_hbm, out_hbm,
              idx0, idx1, idx2, idx3, rows0, rows1, rows2, rows3,
              stg0, stg1,
              isem0, isem1, isem2, isem3,
              gsem0, gsem1, gsem2, gsem3, wsem0, wsem1):
    wid = lax.axis_index("s") * _NC + lax.axis_index("c")
    base = wid * _ROWS_PER_W

    idxs = (idx0, idx1, idx2, idx3)
    rows = (rows0, rows1, rows2, rows3)
    stgs = (stg0, stg1)
    isems = (isem0, isem1, isem2, isem3)
    gsems = (gsem0, gsem1, gsem2, gsem3)
    wsems = (wsem0, wsem1)

    def fire_i(g, b):
        pltpu.make_async_copy(
            idx_hbm.at[pl.ds(base + g * _K, _K)], idxs[b], isems[b]).start()

    def wait_i(b):
        pltpu.make_async_copy(
            idx_hbm.at[pl.ds(0, _K)], idxs[b], isems[b]).wait()

    def fire_g(b):
        pltpu.make_async_copy(
            lut_hbm.at[idxs[b]], rows[b], gsems[b]).start()

    def wait_g(b):
        pltpu.make_async_copy(
            lut_hbm.at[pl.ds(0, _K), :], rows[b], gsems[b]).wait()

    def fire_w(g, b):
        pltpu.make_async_copy(
            stgs[b], out_hbm.at[pl.ds(base + g * _K, _K), :], wsems[b]).start()

    def wait_w(b):
        pltpu.make_async_copy(
            stgs[b], out_hbm.at[pl.ds(0, _K), :], wsems[b]).wait()

    def scale(rb, sb):
        r = rows[rb]
        s = stgs[sb]

        @plsc.parallel_loop(0, _K, step=1, unroll=_UNROLL)
        def body(i):
            for t in range(_D // 16):
                s[i, pl.ds(16 * t, 16)] = r[i, pl.ds(16 * t, 16)] * _SCALE

    for g in range(4):
        fire_i(g, g)
    for g in range(3):
        wait_i(g)
        fire_g(g)
    fire_w(0, 0)
    fire_w(1, 1)

    def quad(q, carry):
        for cc in range(4):
            c = q * 4 + cc
            wait_g(cc)
            wait_w(cc % 2)
            scale(cc, cc % 2)
            fire_w(c, cc % 2)

            @pl.when(c + 4 < _CHUNKS)
            def _():
                fire_i(c + 4, cc)

            @pl.when(c + 3 < _CHUNKS)
            def _():
                wait_i((cc + 3) % 4)
                fire_g((cc + 3) % 4)

        return carry

    lax.fori_loop(0, _CHUNKS // 4, quad, 0)
    wait_w(0)
    wait_w(1)


def kernel(x, lut):
    idx1d = x.reshape(_N)
    lut_pad = jnp.pad(lut, ((0, 0), (0, 128 - _D)))
    run = functools.partial(
        pl.kernel,
        mesh=plsc.VectorSubcoreMesh(core_axis_name="c", subcore_axis_name="s"),
        out_type=jax.ShapeDtypeStruct((_N, _D), jnp.float32),
        scratch_types=(
            [pltpu.VMEM((_K,), jnp.int32)] * 4
            + [pltpu.VMEM((_K, 128), jnp.float32)] * 4
            + [pltpu.VMEM((_K, _D), jnp.float32)] * 2
            + [pltpu.SemaphoreType.DMA] * 10
        ),
        compiler_params=pltpu.CompilerParams(use_tc_tiling_on_sc=True),
    )(_emb_body)
    out = run(idx1d, lut_pad)
    return out.reshape(_B, _L, _D)


# speedup vs baseline: 1.3507x; 1.0001x over previous
import functools
import math

import jax
import jax.numpy as jnp
from jax import lax
from jax.experimental import pallas as pl
from jax.experimental.pallas import tpu as pltpu
from jax.experimental.pallas import tpu_sc as plsc

_D = 64
_B = 4096
_L = 200

_NC = 2
_NS = 16
_NW = _NC * _NS

_N = _B * _L
_ROWS_PER_W = _N // _NW
_K = 128
_CHUNKS = _ROWS_PER_W // _K
_UNROLL = 8
_SCALE = math.sqrt(_D)


def _emb_body(idx_hbm, lut_hbm, out_hbm,
              idx0, idx1, idx2, idx3, rows0, rows1, rows2, rows3,
              stg0, stg1,
              isem0, isem1, isem2, isem3,
              gsem0, gsem1, gsem2, gsem3, wsem0, wsem1):
    wid = lax.axis_index("s") * _NC + lax.axis_index("c")
    base = wid * _ROWS_PER_W

    idxs = (idx0, idx1, idx2, idx3)
    rows = (rows0, rows1, rows2, rows3)
    stgs = (stg0, stg1)
    isems = (isem0, isem1, isem2, isem3)
    gsems = (gsem0, gsem1, gsem2, gsem3)
    wsems = (wsem0, wsem1)

    def fire_i(g, b):
        pltpu.make_async_copy(
            idx_hbm.at[pl.ds(base + g * _K, _K)], idxs[b], isems[b]).start()

    def wait_i(b):
        pltpu.make_async_copy(
            idx_hbm.at[pl.ds(0, _K)], idxs[b], isems[b]).wait()

    def fire_g(b):
        pltpu.make_async_copy(
            lut_hbm.at[idxs[b]], rows[b], gsems[b]).start()

    def wait_g(b):
        pltpu.make_async_copy(
            lut_hbm.at[pl.ds(0, _K), :], rows[b], gsems[b]).wait()

    def fire_w(g, b):
        pltpu.make_async_copy(
            stgs[b], out_hbm.at[pl.ds(base + g * _K, _K), :], wsems[b]).start()

    def wait_w(b):
        pltpu.make_async_copy(
            stgs[b], out_hbm.at[pl.ds(0, _K), :], wsems[b]).wait()

    def scale(rb, sb):
        r = rows[rb]
        s = stgs[sb]

        @plsc.parallel_loop(0, _K, step=1, unroll=_UNROLL)
        def body(i):
            for t in range(_D // 16):
                s[i, pl.ds(16 * t, 16)] = r[i, pl.ds(16 * t, 16)] * _SCALE

    for g in range(4):
        fire_i(g, g)
    for g in range(3):
        wait_i(g)
        fire_g(g)
    fire_w(0, 0)
    fire_w(1, 1)

    def quad(q, carry):
        for cc in range(4):
            c = q * 4 + cc
            wait_g(cc)
            wait_w(cc % 2)
            scale(cc, cc % 2)
            fire_w(c, cc % 2)

            @pl.when(c + 4 < _CHUNKS)
            def _():
                fire_i(c + 4, cc)

            @pl.when(c + 3 < _CHUNKS)
            def _():
                wait_i((cc + 3) % 4)
                fire_g((cc + 3) % 4)

        return carry

    lax.fori_loop(0, _CHUNKS // 4, quad, 0)
    wait_w(0)
    wait_w(1)


def kernel(x, lut):
    idx1d = x.reshape(_N)
    lut_pad = jnp.pad(lut, ((0, 0), (0, 128 - _D)))
    run = functools.partial(
        pl.kernel,
        mesh=plsc.VectorSubcoreMesh(core_axis_name="c", subcore_axis_name="s"),
        out_type=jax.ShapeDtypeStruct((_N, _D), jnp.float32),
        scratch_types=(
            [pltpu.VMEM((_K,), jnp.int32)] * 4
            + [pltpu.VMEM((_K, 128), jnp.float32)] * 4
            + [pltpu.VMEM((_K, _D), jnp.float32)] * 2
            + [pltpu.SemaphoreType.DMA] * 10
        ),
        compiler_params=pltpu.CompilerParams(use_tc_tiling_on_sc=True),
    )(_emb_body)
    out = run(idx1d, lut_pad)
    return out.reshape(_B, _L, _D)
